# baseline (speedup 1.0000x reference)
import jax
import jax.numpy as jnp
from jax import lax
from jax.experimental import pallas as pl
from jax.experimental.pallas import tpu as pltpu
from jax.experimental.pallas import tpu_sc as plsc

N_GRAPHS = 4096
N_NODES = 32
WINDOW = 128
N_EDGES = 194
GRAPHS_PER_BLOCK = 32
ROWS_PER_BLOCK = GRAPHS_PER_BLOCK * N_NODES
GRID = N_GRAPHS // GRAPHS_PER_BLOCK

_LANES = 16
_CELLS = N_NODES * N_NODES
_CHUNKS = (256 + _LANES - 1) // _LANES


def _sc_adj_body(ei_hbm, ew_hbm, out_hbm, src_v, dst_v, ew_v, acc_v, fin_v):
    wid = lax.axis_index("s") * 2 + lax.axis_index("c")

    @pl.when(wid == 0)
    def _():
        pltpu.sync_copy(ei_hbm.at[0], src_v)
        pltpu.sync_copy(ei_hbm.at[1], dst_v)
        pltpu.sync_copy(ew_hbm, ew_v)
        lanes = lax.iota(jnp.int32, _LANES)
        zeros = jnp.zeros((_LANES,), jnp.float32)

        def zero_body(i, _):
            for j in range(8):
                acc_v[pl.ds(i * 8 * _LANES + j * _LANES, _LANES)] = zeros
            return 0

        lax.fori_loop(0, _LANES * _CELLS // (8 * _LANES), zero_body, 0)

        for c in range(_CHUNKS):
            if c * _LANES >= N_EDGES:
                break
            s = src_v[pl.ds(c * _LANES, _LANES)]
            d = dst_v[pl.ds(c * _LANES, _LANES)]
            w = ew_v[pl.ds(c * _LANES, _LANES)]
            flat = d * N_NODES + s + lanes * _CELLS
            if (c + 1) * _LANES <= N_EDGES:
                plsc.addupdate_scatter(acc_v, [flat], w)
            else:
                m = (c * _LANES + lanes) < N_EDGES
                plsc.addupdate_scatter(acc_v, [flat], w, mask=m)

        def red_body(cc, _):
            pos = cc * _LANES + lanes
            tot = jnp.where(pos % (N_NODES + 1) == 0, 1.0, 0.0)

            def lane_body(l, t):
                return t + acc_v[pl.ds(l * _CELLS + cc * _LANES, _LANES)]

            tot = lax.fori_loop(0, _LANES, lane_body, tot)
            fin_v[pl.ds(cc * _LANES, _LANES)] = tot
            return 0

        lax.fori_loop(0, _CELLS // _LANES, red_body, 0)
        pltpu.sync_copy(fin_v, out_hbm)


def _sc_build_adj(edge_index, edge_attr):
    return pl.kernel(
        _sc_adj_body,
        out_type=jax.ShapeDtypeStruct((_CELLS,), jnp.float32),
        mesh=plsc.VectorSubcoreMesh(core_axis_name="c", subcore_axis_name="s"),
        compiler_params=pltpu.CompilerParams(needs_layout_passes=False),
        scratch_types=[
            pltpu.VMEM((256,), jnp.int32),
            pltpu.VMEM((256,), jnp.int32),
            pltpu.VMEM((256,), jnp.float32),
            pltpu.VMEM((_LANES * _CELLS,), jnp.float32),
            pltpu.VMEM((_CELLS,), jnp.float32),
        ],
    )(edge_index, edge_attr)


def _tc_body(x_ref, A_ref, W1_ref, b1_ref, W2_ref, b2_ref, W3_ref, b3_ref,
             W4_ref, b4_ref, cw_ref, cb_ref, fcw_ref, fcb_ref, out_ref):
    f32 = jnp.float32

    A = A_ref[...]
    T = jnp.concatenate([A, A, A, A], axis=0)
    T = jnp.concatenate([T, T, T, T], axis=1)
    rb = lax.broadcasted_iota(jnp.int32, (128, 128), 0) // N_NODES
    cb = lax.broadcasted_iota(jnp.int32, (128, 128), 1) // N_NODES
    A4 = jnp.where(rb == cb, T, 0.0).astype(f32)

    def mix(v):
        outs = []
        for c in range(ROWS_PER_BLOCK // 128):
            blk = v[c * 128:(c + 1) * 128, :]
            outs.append(lax.dot_general(A4, blk, (((1,), (0,)), ((), ())),
                                        preferred_element_type=f32))
        return jnp.concatenate(outs, axis=0)

    xb = x_ref[...]
    a1 = jax.nn.relu(jnp.dot(xb, W1_ref[...], preferred_element_type=f32)
                     + b1_ref[...])
    h1 = jax.nn.relu(jnp.dot(mix(a1), W2_ref[...], preferred_element_type=f32)
                     + b2_ref[...])
    a2 = jax.nn.relu(jnp.dot(h1, W3_ref[...], preferred_element_type=f32)
                     + b3_ref[...])
    h2 = jax.nn.relu(jnp.dot(mix(a2), W4_ref[...], preferred_element_type=f32)
                     + b4_ref[...])

    y = (jnp.dot(h2[:, :64], cw_ref[:, 0:1], preferred_element_type=f32) +
         jnp.dot(h2[:, 64:], cw_ref[:, 1:2], preferred_element_type=f32))
    y = jax.nn.relu(y + cb_ref[0, 0])

    Y = y.reshape(GRAPHS_PER_BLOCK, N_NODES)
    out = jax.nn.sigmoid(jnp.dot(Y, fcw_ref[...], preferred_element_type=f32)
                         + fcb_ref[0, 0])
    out_ref[...] = out


def _fold_conv(W, kern):
    K = kern.shape[0]
    pad_l = (K - 1) // 2
    C, H = W.shape
    acc = jnp.zeros_like(W)
    for k in range(K):
        s = k - pad_l
        if s == 0:
            sh = W
        elif s > 0:
            sh = jnp.concatenate([jnp.zeros((s, H), W.dtype), W[: C - s, :]],
                                 axis=0)
        else:
            sh = jnp.concatenate([W[-s:, :], jnp.zeros((-s, H), W.dtype)],
                                 axis=0)
        acc = acc + kern[k] * sh
    return acc


def _blockdiag2(W):
    K, H = W.shape
    z = jnp.zeros((K, H), W.dtype)
    top = jnp.concatenate([W, z], axis=1)
    bot = jnp.concatenate([z, W], axis=1)
    return jnp.concatenate([top, bot], axis=0)


def kernel(x, edge_index, edge_attr, batch, kern1, W1a, b1a, W2a, b2a,
           kern2, W1b, b1b, W2b, b2b, conv_w, conv_b, fc_w, fc_b):
    del batch
    f32 = jnp.float32

    Ahat = _sc_build_adj(edge_index.astype(jnp.int32),
                         edge_attr.astype(f32)).reshape(N_NODES, N_NODES)

    W1cat = _blockdiag2(_fold_conv(W1a, kern1))
    b1cat = jnp.tile(b1a, 2).reshape(1, 32)
    W2cat = _blockdiag2(W2a)
    b2cat = jnp.tile(b2a, 2).reshape(1, 128)
    W3cat = _blockdiag2(_fold_conv(W1b, kern2))
    b3cat = jnp.tile(b1b, 2).reshape(1, 16)
    W4cat = _blockdiag2(W2b)
    b4cat = jnp.tile(b2b, 2).reshape(1, 128)
    cw = conv_w[0].astype(f32)
    cb = conv_b.reshape(1, 1).astype(f32)
    fcb = fc_b.reshape(1, 1).astype(f32)

    vspec = lambda shape: pl.BlockSpec(shape, lambda i: (0, 0))
    sspec = lambda shape: pl.BlockSpec(shape, lambda i: (0, 0),
                                       memory_space=pltpu.SMEM)

    return pl.pallas_call(
        _tc_body,
        grid=(GRID,),
        in_specs=[
            pl.BlockSpec((ROWS_PER_BLOCK, 2 * WINDOW), lambda i: (i, 0)),
            vspec((N_NODES, N_NODES)),
            vspec((256, 32)),
            vspec((1, 32)),
            vspec((32, 128)),
            vspec((1, 128)),
            vspec((128, 16)),
            vspec((1, 16)),
            vspec((16, 128)),
            vspec((1, 128)),
            vspec((64, 2)),
            sspec((1, 1)),
            vspec((N_NODES, 1)),
            sspec((1, 1)),
        ],
        out_specs=pl.BlockSpec((GRAPHS_PER_BLOCK, 1), lambda i: (i, 0)),
        out_shape=jax.ShapeDtypeStruct((N_GRAPHS, 1), f32),
        compiler_params=pltpu.CompilerParams(
            dimension_semantics=("parallel",)),
    )(x, Ahat, W1cat, b1cat, W2cat, b2cat, W3cat, b3cat, W4cat, b4cat,
      cw, cb, fc_w, fcb)


# speedup vs baseline: 1.2689x; 1.2689x over previous
import jax
import jax.numpy as jnp
from jax import lax
from jax.experimental import pallas as pl
from jax.experimental.pallas import tpu as pltpu
from jax.experimental.pallas import tpu_sc as plsc

N_GRAPHS = 4096
N_NODES = 32
WINDOW = 128
N_EDGES = 194
GRAPHS_PER_BLOCK = 32
ROWS_PER_BLOCK = GRAPHS_PER_BLOCK * N_NODES
GRID = N_GRAPHS // GRAPHS_PER_BLOCK

_LANES = 16
_CELLS = N_NODES * N_NODES
_CHUNKS = (256 + _LANES - 1) // _LANES


def _sc_adj_body(ei_hbm, ew_hbm, out_hbm, src_v, dst_v, ew_v, acc_v, fin_v):
    wid = lax.axis_index("s") * 2 + lax.axis_index("c")

    @pl.when(wid == 0)
    def _():
        pltpu.sync_copy(ei_hbm.at[0], src_v)
        pltpu.sync_copy(ei_hbm.at[1], dst_v)
        pltpu.sync_copy(ew_hbm, ew_v)
        lanes = lax.iota(jnp.int32, _LANES)
        zeros = jnp.zeros((_LANES,), jnp.float32)

        def zero_body(i, _):
            for j in range(8):
                acc_v[pl.ds(i * 8 * _LANES + j * _LANES, _LANES)] = zeros
            return 0

        lax.fori_loop(0, _LANES * _CELLS // (8 * _LANES), zero_body, 0)

        for c in range(_CHUNKS):
            if c * _LANES >= N_EDGES:
                break
            s = src_v[pl.ds(c * _LANES, _LANES)]
            d = dst_v[pl.ds(c * _LANES, _LANES)]
            w = ew_v[pl.ds(c * _LANES, _LANES)]
            flat = d * N_NODES + s + lanes * _CELLS
            if (c + 1) * _LANES <= N_EDGES:
                plsc.addupdate_scatter(acc_v, [flat], w)
            else:
                m = (c * _LANES + lanes) < N_EDGES
                plsc.addupdate_scatter(acc_v, [flat], w, mask=m)

        def red_body(cc, _):
            pos = cc * _LANES + lanes
            tot = jnp.where(pos % (N_NODES + 1) == 0, 1.0, 0.0)

            def lane_body(l, t):
                return t + acc_v[pl.ds(l * _CELLS + cc * _LANES, _LANES)]

            tot = lax.fori_loop(0, _LANES, lane_body, tot)
            fin_v[pl.ds(cc * _LANES, _LANES)] = tot
            return 0

        lax.fori_loop(0, _CELLS // _LANES, red_body, 0)
        pltpu.sync_copy(fin_v, out_hbm)


def _sc_build_adj(edge_index, edge_attr):
    return pl.kernel(
        _sc_adj_body,
        out_type=jax.ShapeDtypeStruct((_CELLS,), jnp.float32),
        mesh=plsc.VectorSubcoreMesh(core_axis_name="c", subcore_axis_name="s"),
        compiler_params=pltpu.CompilerParams(needs_layout_passes=False),
        scratch_types=[
            pltpu.VMEM((256,), jnp.int32),
            pltpu.VMEM((256,), jnp.int32),
            pltpu.VMEM((256,), jnp.float32),
            pltpu.VMEM((_LANES * _CELLS,), jnp.float32),
            pltpu.VMEM((_CELLS,), jnp.float32),
        ],
    )(edge_index, edge_attr)


def _fold_conv(W, kref, K):
    pad_l = (K - 1) // 2
    C, H = W.shape
    acc = W * kref[0, pad_l]
    for k in range(K):
        s = k - pad_l
        if s == 0:
            continue
        if s > 0:
            sh = jnp.concatenate([jnp.zeros((s, H), W.dtype), W[: C - s, :]],
                                 axis=0)
        else:
            sh = jnp.concatenate([W[-s:, :], jnp.zeros((-s, H), W.dtype)],
                                 axis=0)
        acc = acc + kref[0, k] * sh
    return acc


def _blockdiag2(W):
    K, H = W.shape
    z = jnp.zeros((K, H), W.dtype)
    top = jnp.concatenate([W, z], axis=1)
    bot = jnp.concatenate([z, W], axis=1)
    return jnp.concatenate([top, bot], axis=0)


def _tc_body(x_ref, A_ref, k1_ref, W1a_ref, b1a_ref, W2a_ref, b2a_ref,
             k2_ref, W1b_ref, b1b_ref, W2b_ref, b2b_ref, cw_ref, cb_ref,
             fcw_ref, fcb_ref, out_ref,
             W1s, W2s, W3s, W4s, A4s, b1s, b2s, b3s, b4s, cws):
    f32 = jnp.float32

    @pl.when(pl.program_id(0) == 0)
    def _prep():
        W1s[...] = _blockdiag2(_fold_conv(W1a_ref[...], k1_ref, 15))
        W2s[...] = _blockdiag2(W2a_ref[...])
        W3s[...] = _blockdiag2(_fold_conv(W1b_ref[...], k2_ref, 16))
        W4s[...] = _blockdiag2(W2b_ref[...])
        b1s[...] = jnp.concatenate([b1a_ref[...], b1a_ref[...]], axis=1)
        b2s[...] = jnp.concatenate([b2a_ref[...], b2a_ref[...]], axis=1)
        b3s[...] = jnp.concatenate([b1b_ref[...], b1b_ref[...]], axis=1)
        b4s[...] = jnp.concatenate([b2b_ref[...], b2b_ref[...]], axis=1)
        cws[...] = jnp.concatenate([cw_ref[:, 0:1], cw_ref[:, 1:2]], axis=0)
        A = A_ref[...]
        T = jnp.concatenate([A, A, A, A], axis=0)
        T = jnp.concatenate([T, T, T, T], axis=1)
        rb = lax.broadcasted_iota(jnp.int32, (128, 128), 0) // N_NODES
        cb = lax.broadcasted_iota(jnp.int32, (128, 128), 1) // N_NODES
        A4s[...] = jnp.where(rb == cb, T, 0.0)

    A4 = A4s[...]

    def mix(v):
        outs = []
        for c in range(ROWS_PER_BLOCK // 128):
            blk = v[c * 128:(c + 1) * 128, :]
            outs.append(lax.dot_general(A4, blk, (((1,), (0,)), ((), ())),
                                        preferred_element_type=f32))
        return jnp.concatenate(outs, axis=0)

    xb = x_ref[...]
    a1 = jax.nn.relu(jnp.dot(xb, W1s[...], preferred_element_type=f32)
                     + b1s[...])
    h1 = jax.nn.relu(jnp.dot(mix(a1), W2s[...], preferred_element_type=f32)
                     + b2s[...])
    a2 = jax.nn.relu(jnp.dot(h1, W3s[...], preferred_element_type=f32)
                     + b3s[...])
    h2 = jax.nn.relu(jnp.dot(mix(a2), W4s[...], preferred_element_type=f32)
                     + b4s[...])

    y = jnp.dot(h2, cws[...], preferred_element_type=f32)
    y = jax.nn.relu(y + cb_ref[0, 0])
    Y = y.reshape(GRAPHS_PER_BLOCK, N_NODES)
    out = jax.nn.sigmoid(jnp.dot(Y, fcw_ref[...], preferred_element_type=f32)
                         + fcb_ref[0, 0])
    out_ref[...] = out


def kernel(x, edge_index, edge_attr, batch, kern1, W1a, b1a, W2a, b2a,
           kern2, W1b, b1b, W2b, b2b, conv_w, conv_b, fc_w, fc_b):
    del batch
    f32 = jnp.float32

    Ahat = _sc_build_adj(edge_index.astype(jnp.int32),
                         edge_attr.astype(f32)).reshape(N_NODES, N_NODES)

    vspec = lambda shape: pl.BlockSpec(shape, lambda i: (0, 0))
    sspec = lambda shape: pl.BlockSpec(shape, lambda i: (0, 0),
                                       memory_space=pltpu.SMEM)

    return pl.pallas_call(
        _tc_body,
        grid=(GRID,),
        in_specs=[
            pl.BlockSpec((ROWS_PER_BLOCK, 2 * WINDOW), lambda i: (i, 0)),
            vspec((N_NODES, N_NODES)),
            sspec((1, 15)),
            vspec((WINDOW, 16)),
            vspec((1, 16)),
            vspec((16, 64)),
            vspec((1, 64)),
            sspec((1, 16)),
            vspec((64, 8)),
            vspec((1, 8)),
            vspec((8, 64)),
            vspec((1, 64)),
            vspec((64, 2)),
            sspec((1, 1)),
            vspec((N_NODES, 1)),
            sspec((1, 1)),
        ],
        out_specs=pl.BlockSpec((GRAPHS_PER_BLOCK, 1), lambda i: (i, 0)),
        out_shape=jax.ShapeDtypeStruct((N_GRAPHS, 1), f32),
        scratch_shapes=[
            pltpu.VMEM((256, 32), f32),
            pltpu.VMEM((32, 128), f32),
            pltpu.VMEM((128, 16), f32),
            pltpu.VMEM((16, 128), f32),
            pltpu.VMEM((128, 128), f32),
            pltpu.VMEM((1, 32), f32),
            pltpu.VMEM((1, 128), f32),
            pltpu.VMEM((1, 16), f32),
            pltpu.VMEM((1, 128), f32),
            pltpu.VMEM((128, 1), f32),
        ],
        compiler_params=pltpu.CompilerParams(
            dimension_semantics=("arbitrary",)),
    )(x, Ahat, kern1.reshape(1, 15), W1a, b1a.reshape(1, 16), W2a,
      b2a.reshape(1, 64), kern2.reshape(1, 16), W1b, b1b.reshape(1, 8),
      W2b, b2b.reshape(1, 64), conv_w[0], conv_b.reshape(1, 1), fc_w,
      fc_b.reshape(1, 1))


# speedup vs baseline: 2.5226x; 1.9881x over previous
import jax
import jax.numpy as jnp
from jax import lax
from jax.experimental import pallas as pl
from jax.experimental.pallas import tpu as pltpu
from jax.experimental.pallas import tpu_sc as plsc

N_GRAPHS = 4096
N_NODES = 32
WINDOW = 128
N_EDGES = 194
GRAPHS_PER_BLOCK = 128
ROWS_PER_BLOCK = GRAPHS_PER_BLOCK * N_NODES
GRID = N_GRAPHS // GRAPHS_PER_BLOCK

_LANES = 16
_CELLS = N_NODES * N_NODES
_CHUNKS = (256 + _LANES - 1) // _LANES


def _sc_adj_body(ei_hbm, ew_hbm, out_hbm, src_v, dst_v, ew_v, acc_v, fin_v):
    wid = lax.axis_index("s") * 2 + lax.axis_index("c")

    @pl.when(wid == 0)
    def _():
        pltpu.sync_copy(ei_hbm.at[0], src_v)
        pltpu.sync_copy(ei_hbm.at[1], dst_v)
        pltpu.sync_copy(ew_hbm, ew_v)
        lanes = lax.iota(jnp.int32, _LANES)
        zeros = jnp.zeros((_LANES,), jnp.float32)

        def zero_body(i, _):
            for j in range(8):
                acc_v[pl.ds(i * 8 * _LANES + j * _LANES, _LANES)] = zeros
            return 0

        lax.fori_loop(0, _LANES * _CELLS // (8 * _LANES), zero_body, 0)

        for c in range(_CHUNKS):
            if c * _LANES >= N_EDGES:
                break
            s = src_v[pl.ds(c * _LANES, _LANES)]
            d = dst_v[pl.ds(c * _LANES, _LANES)]
            w = ew_v[pl.ds(c * _LANES, _LANES)]
            flat = s * N_NODES + d + lanes * _CELLS
            if (c + 1) * _LANES <= N_EDGES:
                plsc.addupdate_scatter(acc_v, [flat], w)
            else:
                m = (c * _LANES + lanes) < N_EDGES
                plsc.addupdate_scatter(acc_v, [flat], w, mask=m)

        def red_body(cc, _):
            pos = cc * _LANES + lanes
            tot = jnp.where(pos % (N_NODES + 1) == 0, 1.0, 0.0)

            def lane_body(l, t):
                return t + acc_v[pl.ds(l * _CELLS + cc * _LANES, _LANES)]

            tot = lax.fori_loop(0, _LANES, lane_body, tot)
            fin_v[pl.ds(cc * _LANES, _LANES)] = tot
            return 0

        lax.fori_loop(0, _CELLS // _LANES, red_body, 0)
        pltpu.sync_copy(fin_v, out_hbm)


def _sc_build_adj(edge_index, edge_attr):
    return pl.kernel(
        _sc_adj_body,
        out_type=jax.ShapeDtypeStruct((_CELLS,), jnp.float32),
        mesh=plsc.VectorSubcoreMesh(core_axis_name="c", subcore_axis_name="s"),
        compiler_params=pltpu.CompilerParams(needs_layout_passes=False),
        scratch_types=[
            pltpu.VMEM((256,), jnp.int32),
            pltpu.VMEM((256,), jnp.int32),
            pltpu.VMEM((256,), jnp.float32),
            pltpu.VMEM((_LANES * _CELLS,), jnp.float32),
            pltpu.VMEM((_CELLS,), jnp.float32),
        ],
    )(edge_index, edge_attr)


def _fold_conv(W, kref, K):
    pad_l = (K - 1) // 2
    C, H = W.shape
    acc = W * kref[0, pad_l]
    for k in range(K):
        s = k - pad_l
        if s == 0:
            continue
        if s > 0:
            sh = jnp.concatenate([jnp.zeros((s, H), W.dtype), W[: C - s, :]],
                                 axis=0)
        else:
            sh = jnp.concatenate([W[-s:, :], jnp.zeros((-s, H), W.dtype)],
                                 axis=0)
        acc = acc + kref[0, k] * sh
    return acc


def _fold_conv_cols(WT, kref, K):
    pad_l = (K - 1) // 2
    H, C = WT.shape
    acc = WT * kref[0, pad_l]
    for k in range(K):
        s = k - pad_l
        if s == 0:
            continue
        if s > 0:
            sh = jnp.concatenate([jnp.zeros((H, s), WT.dtype), WT[:, : C - s]],
                                 axis=1)
        else:
            sh = jnp.concatenate([WT[:, -s:], jnp.zeros((H, -s), WT.dtype)],
                                 axis=1)
        acc = acc + kref[0, k] * sh
    return acc


def _blockdiag2(W):
    K, H = W.shape
    z = jnp.zeros((K, H), W.dtype)
    top = jnp.concatenate([W, z], axis=1)
    bot = jnp.concatenate([z, W], axis=1)
    return jnp.concatenate([top, bot], axis=0)


def _tc_body(x_ref, AT_ref, k1_ref, W1a_ref, b1a_ref, W2aT_ref, b2a_ref,
             k2_ref, W1bT_ref, b1b_ref, W2bT_ref, b2b_ref, cwT_ref, cb_ref,
             fcw_ref, fcb_ref, out_ref,
             W1s, W2Ts, W3Ts, W4Ts, A4Ts, b1s, b2s, b3s, b4s, Fs):
    f32 = jnp.float32

    @pl.when(pl.program_id(0) == 0)
    def _prep():
        W1s[...] = _blockdiag2(_fold_conv(W1a_ref[...], k1_ref, 15))
        W2Ts[...] = _blockdiag2(W2aT_ref[...])
        W3Ts[...] = _blockdiag2(_fold_conv_cols(W1bT_ref[...], k2_ref, 16))
        W4Ts[...] = _blockdiag2(W2bT_ref[...])
        b1s[...] = jnp.concatenate([b1a_ref[...], b1a_ref[...]], axis=0)
        b2s[...] = jnp.concatenate([b2a_ref[...], b2a_ref[...]], axis=0)
        b3s[...] = jnp.concatenate([b1b_ref[...], b1b_ref[...]], axis=0)
        b4s[...] = jnp.concatenate([b2b_ref[...], b2b_ref[...]], axis=0)
        AT = AT_ref[...]
        T = jnp.concatenate([AT, AT, AT, AT], axis=0)
        T = jnp.concatenate([T, T, T, T], axis=1)
        rb = lax.broadcasted_iota(jnp.int32, (128, 128), 0) // N_NODES
        cb = lax.broadcasted_iota(jnp.int32, (128, 128), 1) // N_NODES
        A4Ts[...] = jnp.where(rb == cb, T, 0.0)
        fcw_full = jnp.concatenate([fcw_ref[...]] * GRAPHS_PER_BLOCK, axis=0)
        rf = lax.broadcasted_iota(jnp.int32, (ROWS_PER_BLOCK,
                                              GRAPHS_PER_BLOCK), 0) // N_NODES
        cf = lax.broadcasted_iota(jnp.int32, (ROWS_PER_BLOCK,
                                              GRAPHS_PER_BLOCK), 1)
        Fs[...] = jnp.where(rf == cf, fcw_full, 0.0)

    A4T = A4Ts[...]

    def mixT(v):
        outs = []
        for c in range(ROWS_PER_BLOCK // 128):
            blk = v[:, c * 128:(c + 1) * 128]
            outs.append(jnp.dot(blk, A4T, preferred_element_type=f32))
        return jnp.concatenate(outs, axis=1)

    xb = x_ref[...]
    raw1 = lax.dot_general(W1s[...], xb, (((0,), (1,)), ((), ())),
                           preferred_element_type=f32)
    a1 = jax.nn.relu(raw1 + b1s[...])
    h1 = jax.nn.relu(jnp.dot(W2Ts[...], mixT(a1), preferred_element_type=f32)
                     + b2s[...])
    a2 = jax.nn.relu(jnp.dot(W3Ts[...], h1, preferred_element_type=f32)
                     + b3s[...])
    h2 = jax.nn.relu(jnp.dot(W4Ts[...], mixT(a2), preferred_element_type=f32)
                     + b4s[...])

    y = jnp.dot(cwT_ref[...], h2, preferred_element_type=f32)
    y = jax.nn.relu(y + cb_ref[0, 0])
    out = jax.nn.sigmoid(jnp.dot(y, Fs[...], preferred_element_type=f32)
                         + fcb_ref[0, 0])
    out_ref[...] = out.reshape(1, 1, GRAPHS_PER_BLOCK)


def kernel(x, edge_index, edge_attr, batch, kern1, W1a, b1a, W2a, b2a,
           kern2, W1b, b1b, W2b, b2b, conv_w, conv_b, fc_w, fc_b):
    del batch
    f32 = jnp.float32

    AhatT = _sc_build_adj(edge_index.astype(jnp.int32),
                          edge_attr.astype(f32)).reshape(N_NODES, N_NODES)

    vspec = lambda shape: pl.BlockSpec(shape, lambda i: (0, 0))
    sspec = lambda shape: pl.BlockSpec(shape, lambda i: (0, 0),
                                       memory_space=pltpu.SMEM)

    return pl.pallas_call(
        _tc_body,
        grid=(GRID,),
        in_specs=[
            pl.BlockSpec((ROWS_PER_BLOCK, 2 * WINDOW), lambda i: (i, 0)),
            vspec((N_NODES, N_NODES)),
            sspec((1, 15)),
            vspec((WINDOW, 16)),
            vspec((16, 1)),
            vspec((64, 16)),
            vspec((64, 1)),
            sspec((1, 16)),
            vspec((8, 64)),
            vspec((8, 1)),
            vspec((64, 8)),
            vspec((64, 1)),
            vspec((1, 128)),
            sspec((1, 1)),
            vspec((N_NODES, 1)),
            sspec((1, 1)),
        ],
        out_specs=pl.BlockSpec((1, 1, GRAPHS_PER_BLOCK), lambda i: (i, 0, 0)),
        out_shape=jax.ShapeDtypeStruct((GRID, 1, GRAPHS_PER_BLOCK), f32),
        scratch_shapes=[
            pltpu.VMEM((256, 32), f32),
            pltpu.VMEM((128, 32), f32),
            pltpu.VMEM((16, 128), f32),
            pltpu.VMEM((128, 16), f32),
            pltpu.VMEM((128, 128), f32),
            pltpu.VMEM((32, 1), f32),
            pltpu.VMEM((128, 1), f32),
            pltpu.VMEM((16, 1), f32),
            pltpu.VMEM((128, 1), f32),
            pltpu.VMEM((ROWS_PER_BLOCK, GRAPHS_PER_BLOCK), f32),
        ],
        compiler_params=pltpu.CompilerParams(
            dimension_semantics=("arbitrary",)),
    )(x, AhatT, kern1.reshape(1, 15), W1a, b1a.reshape(16, 1), W2a.T,
      b2a.reshape(64, 1), kern2.reshape(1, 16), W1b.T, b1b.reshape(8, 1),
      W2b.T, b2b.reshape(64, 1), conv_w[0].T.reshape(1, 128),
      conv_b.reshape(1, 1), fc_w, fc_b.reshape(1, 1)).reshape(N_GRAPHS, 1)
